# Initial kernel scaffold; baseline (speedup 1.0000x reference)
import jax
import jax.numpy as jnp
from jax.experimental import pallas as pl


def kernel(h, gate_w, Wg, Wu, Wd, sWg, sWu, sWd):
    raise NotImplementedError("write your pallas kernel here")


# speedup vs baseline: 1.9089x; 1.9089x over previous
import jax
import jax.numpy as jnp
from jax.experimental import pallas as pl
from jax.experimental.pallas import tpu as pltpu

T = 4096
D = 1024
E = 16
K = 2
DFF = 512
NSH = 2
BB = 512


def _moe_dense_kernel(h_ref, gw_ref, wg_ref, wu_ref, wd_ref,
                      swg_ref, swu_ref, swd_ref,
                      out_ref, acc_ref, wdense_ref):
    e = pl.program_id(1)
    x32 = h_ref[...]
    x = x32.astype(jnp.bfloat16)
    iota = jax.lax.broadcasted_iota(jnp.int32, (BB, E), 1)

    @pl.when(e == 0)
    def _gate_and_shared():
        logits = jnp.dot(x32, gw_ref[...].T, preferred_element_type=jnp.float32)
        m = jnp.max(logits, axis=1, keepdims=True)
        ex = jnp.exp(logits - m)
        s = ex / jnp.sum(ex, axis=1, keepdims=True)
        e1 = jnp.argmax(s, axis=1)
        m1 = iota == e1[:, None]
        s2 = jnp.where(m1, -jnp.inf, s)
        e2 = jnp.argmax(s2, axis=1)
        m2 = iota == e2[:, None]
        wdense_ref[...] = jnp.where(m1 | m2, s, 0.0)
        g = jnp.dot(x, swg_ref[...], preferred_element_type=jnp.float32)
        u = jnp.dot(x, swu_ref[...], preferred_element_type=jnp.float32)
        a = (jax.nn.silu(g) * u).astype(jnp.bfloat16)
        acc_ref[...] = jnp.dot(a, swd_ref[...], preferred_element_type=jnp.float32)

    g = jnp.dot(x, wg_ref[0], preferred_element_type=jnp.float32)
    u = jnp.dot(x, wu_ref[0], preferred_element_type=jnp.float32)
    a = (jax.nn.silu(g) * u).astype(jnp.bfloat16)
    o = jnp.dot(a, wd_ref[0], preferred_element_type=jnp.float32)
    w_col = jnp.sum(jnp.where(iota == e, wdense_ref[...], 0.0), axis=1,
                    keepdims=True)
    acc_ref[...] += o * w_col

    @pl.when(e == E - 1)
    def _write():
        out_ref[...] = acc_ref[...]


def kernel(h, gate_w, Wg, Wu, Wd, sWg, sWu, sWd):
    Wg16 = Wg.astype(jnp.bfloat16)
    Wu16 = Wu.astype(jnp.bfloat16)
    Wd16 = Wd.astype(jnp.bfloat16)
    sWg16 = sWg.astype(jnp.bfloat16)
    sWu16 = sWu.astype(jnp.bfloat16)
    sWd16 = sWd.astype(jnp.bfloat16)

    grid = (T // BB, E)
    out = pl.pallas_call(
        _moe_dense_kernel,
        grid=grid,
        in_specs=[
            pl.BlockSpec((BB, D), lambda t, e: (t, 0)),
            pl.BlockSpec((E, D), lambda t, e: (0, 0)),
            pl.BlockSpec((1, D, DFF), lambda t, e: (e, 0, 0)),
            pl.BlockSpec((1, D, DFF), lambda t, e: (e, 0, 0)),
            pl.BlockSpec((1, DFF, D), lambda t, e: (e, 0, 0)),
            pl.BlockSpec((D, NSH * DFF), lambda t, e: (0, 0)),
            pl.BlockSpec((D, NSH * DFF), lambda t, e: (0, 0)),
            pl.BlockSpec((NSH * DFF, D), lambda t, e: (0, 0)),
        ],
        out_specs=pl.BlockSpec((BB, D), lambda t, e: (t, 0)),
        out_shape=jax.ShapeDtypeStruct((T, D), jnp.float32),
        scratch_shapes=[
            pltpu.VMEM((BB, D), jnp.float32),
            pltpu.VMEM((BB, E), jnp.float32),
        ],
    )(h, gate_w, Wg16, Wu16, Wd16, sWg16, sWu16, sWd16)
    return out
